# Initial kernel scaffold; baseline (speedup 1.0000x reference)
import jax
import jax.numpy as jnp
from jax.experimental import pallas as pl


def kernel(t, emb, change, rel_emb, W_in, W_out, W_loop, loop_rel, W_jump, jump_weight, edge_w_jump, edge_index, edge_type, edge_id_jump):
    raise NotImplementedError("write your pallas kernel here")


# speedup vs baseline: 1.9281x; 1.9281x over previous
import functools

import jax
import jax.numpy as jnp
from jax import lax
from jax.experimental import pallas as pl
from jax.experimental.pallas import tpu as pltpu
from jax.experimental.pallas import tpu_sc as plsc

_N = 10000
_D = 128
_E = 320000
_EJ = 160000
_HALF = _E // 2
_NC = 2
_NS = 16
_K = 40
_CCONV = _HALF // (_K * _NS)
_CJ = _EJ // (_K * _NC * _NS)
_RSTEP = 624
_RCNT = 640


def _sc_body(emb, negrel, srcc, typc, dstc, srcj, dstj, wbj,
             acc_in, acc_out, deg_in, deg_out, accj0, accj1,
             acc_sp, sidx, tidx, didx, rows_v, ones_v, wb_v, sem1):
  c = lax.axis_index("c")
  s = lax.axis_index("s")
  row0 = pl.multiple_of(s * _RSTEP, 8)

  zero16 = jnp.zeros((16,), jnp.float32)
  lane = lax.iota(jnp.int32, 16)
  one0 = jnp.where(lane == 0, 1.0, 0.0).astype(jnp.float32)

  def _init_row(r, carry):
    ones_v[r, pl.ds(0, 16)] = one0
    for j in range(1, _D // 16):
      ones_v[r, pl.ds(16 * j, 16)] = zero16
    for j in range(_D // 16):
      rows_v[r, pl.ds(16 * j, 16)] = zero16
    return carry

  def _zero_rows(r, carry):
    for j in range(_D // 16):
      rows_v[r, pl.ds(16 * j, 16)] = zero16
    return carry

  def _zero_acc():
    for b in range(_RCNT // _K):
      pltpu.sync_copy(rows_v, acc_sp.at[pl.ds(row0 + b * _K, _K)])

  lax.fori_loop(0, _K, _init_row, None)
  _zero_acc()
  plsc.subcore_barrier()

  ebase = c * _HALF + s * (_K * _CCONV)

  def conv_chunk(i, carry):
    off = pl.multiple_of(ebase + i * _K, 8)
    pltpu.sync_copy(srcc.at[pl.ds(off, _K)], sidx)
    pltpu.sync_copy(typc.at[pl.ds(off, _K)], tidx)
    pltpu.sync_copy(dstc.at[pl.ds(off, _K)], didx)
    pltpu.async_copy(emb.at[sidx], rows_v, sem1).wait()
    pltpu.sync_copy(rows_v, acc_sp.at[didx], add=True)
    pltpu.async_copy(negrel.at[tidx], rows_v, sem1).wait()
    pltpu.sync_copy(rows_v, acc_sp.at[didx], add=True)
    return carry

  lax.fori_loop(0, _CCONV, conv_chunk, None)
  plsc.subcore_barrier()

  @pl.when(c == 0)
  def _dump_in():
    pltpu.sync_copy(acc_sp.at[pl.ds(row0, _RCNT)],
                    acc_in.at[pl.ds(row0, _RCNT)])

  @pl.when(c == 1)
  def _dump_out():
    pltpu.sync_copy(acc_sp.at[pl.ds(row0, _RCNT)],
                    acc_out.at[pl.ds(row0, _RCNT)])

  plsc.subcore_barrier()

  lax.fori_loop(0, _K, _zero_rows, None)
  _zero_acc()
  plsc.subcore_barrier()

  def deg_chunk(i, carry):
    off = pl.multiple_of(ebase + i * _K, 8)
    pltpu.sync_copy(dstc.at[pl.ds(off, _K)], didx)
    pltpu.sync_copy(ones_v, acc_sp.at[didx], add=True)
    return carry

  lax.fori_loop(0, _CCONV, deg_chunk, None)
  plsc.subcore_barrier()

  @pl.when(c == 0)
  def _dump_deg_in():
    pltpu.sync_copy(acc_sp.at[pl.ds(row0, _RCNT)],
                    deg_in.at[pl.ds(row0, _RCNT)])

  @pl.when(c == 1)
  def _dump_deg_out():
    pltpu.sync_copy(acc_sp.at[pl.ds(row0, _RCNT)],
                    deg_out.at[pl.ds(row0, _RCNT)])

  plsc.subcore_barrier()

  _zero_acc()
  plsc.subcore_barrier()

  jbase = (c * _NS + s) * (_K * _CJ)

  def jump_chunk(i, carry):
    off = pl.multiple_of(jbase + i * _K, 8)
    pltpu.sync_copy(srcj.at[pl.ds(off, _K)], sidx)
    pltpu.sync_copy(dstj.at[pl.ds(off, _K)], didx)
    pltpu.sync_copy(wbj.at[pl.ds(off, _K)], wb_v)
    pltpu.async_copy(emb.at[sidx], rows_v, sem1).wait()

    def scale_row(r, carry2):
      wvec = wb_v[r, :]
      for j in range(_D // 16):
        rows_v[r, pl.ds(16 * j, 16)] = rows_v[r, pl.ds(16 * j, 16)] * wvec
      return carry2

    lax.fori_loop(0, _K, scale_row, None)
    pltpu.sync_copy(rows_v, acc_sp.at[didx], add=True)
    return carry

  lax.fori_loop(0, _CJ, jump_chunk, None)
  plsc.subcore_barrier()

  @pl.when(c == 0)
  def _dump_j0():
    pltpu.sync_copy(acc_sp.at[pl.ds(row0, _RCNT)],
                    accj0.at[pl.ds(row0, _RCNT)])

  @pl.when(c == 1)
  def _dump_j1():
    pltpu.sync_copy(acc_sp.at[pl.ds(row0, _RCNT)],
                    accj1.at[pl.ds(row0, _RCNT)])


_sc_gather_scatter = functools.partial(
    pl.kernel,
    out_type=(
        jax.ShapeDtypeStruct((_N, _D), jnp.float32),
        jax.ShapeDtypeStruct((_N, _D), jnp.float32),
        jax.ShapeDtypeStruct((_N, _D), jnp.float32),
        jax.ShapeDtypeStruct((_N, _D), jnp.float32),
        jax.ShapeDtypeStruct((_N, _D), jnp.float32),
        jax.ShapeDtypeStruct((_N, _D), jnp.float32),
    ),
    mesh=plsc.VectorSubcoreMesh(core_axis_name="c", subcore_axis_name="s"),
    scratch_types=[
        pltpu.VMEM_SHARED((_N, _D), jnp.float32),
        pltpu.VMEM((_K,), jnp.int32),
        pltpu.VMEM((_K,), jnp.int32),
        pltpu.VMEM((_K,), jnp.int32),
        pltpu.VMEM((_K, _D), jnp.float32),
        pltpu.VMEM((_K, _D), jnp.float32),
        pltpu.VMEM((_K, 16), jnp.float32),
        pltpu.SemaphoreType.DMA,
    ],
)(_sc_body)


_BLK = 2000


def _tc_body(acc_in, acc_out, deg_in, deg_out, aj0, aj1, emb, lrel,
             w_in, w_out, w_loop, w_jump, jw, out):
  ni = 1.0 / jnp.maximum(deg_in[:, 0:1], 1.0)
  no = 1.0 / jnp.maximum(deg_out[:, 0:1], 1.0)
  h = (jnp.dot(acc_in[...] * ni, w_in[...], preferred_element_type=jnp.float32)
       + jnp.dot(acc_out[...] * no, w_out[...],
                 preferred_element_type=jnp.float32)
       + jnp.dot(emb[...] - lrel[...], w_loop[...],
                 preferred_element_type=jnp.float32)) * (1.0 / 3.0)
  jr = jnp.dot(aj0[...] + aj1[...], w_jump[...],
               preferred_element_type=jnp.float32)
  out[...] = jnp.tanh(h) + jw[0, 0] * jr


def _tc_combine(acc_in, acc_out, deg_in, deg_out, aj0, aj1, emb, lrel,
                w_in, w_out, w_loop, w_jump, jw):
  row = pl.BlockSpec((_BLK, _D), lambda i: (i, 0))
  mat = pl.BlockSpec((_D, _D), lambda i: (0, 0))
  return pl.pallas_call(
      _tc_body,
      grid=(_N // _BLK,),
      in_specs=[row, row, row, row, row, row, row,
                pl.BlockSpec((1, _D), lambda i: (0, 0)),
                mat, mat, mat, mat,
                pl.BlockSpec((1, 1), lambda i: (0, 0))],
      out_specs=row,
      out_shape=jax.ShapeDtypeStruct((_N, _D), jnp.float32),
  )(acc_in, acc_out, deg_in, deg_out, aj0, aj1, emb, lrel,
    w_in, w_out, w_loop, w_jump, jw)


def kernel(t, emb, change, rel_emb, W_in, W_out, W_loop, loop_rel, W_jump,
           jump_weight, edge_w_jump, edge_index, edge_type, edge_id_jump):
  srcc = edge_index[0]
  dstc = edge_index[1]
  srcj = edge_id_jump[0]
  dstj = edge_id_jump[1]
  wbj = jnp.broadcast_to(edge_w_jump, (_EJ, 16))
  negrel = -rel_emb
  acc_in, acc_out, deg_in, deg_out, aj0, aj1 = _sc_gather_scatter(
      emb, negrel, srcc, edge_type, dstc, srcj, dstj, wbj)
  dchange = _tc_combine(acc_in, acc_out, deg_in, deg_out, aj0, aj1,
                        emb, loop_rel, W_in, W_out, W_loop, W_jump,
                        jump_weight.reshape(1, 1))
  return (change, dchange)


# speedup vs baseline: 4.1785x; 2.1672x over previous
import functools

import jax
import jax.numpy as jnp
from jax import lax
from jax.experimental import pallas as pl
from jax.experimental.pallas import tpu as pltpu
from jax.experimental.pallas import tpu_sc as plsc

_N = 10000
_D = 128
_E = 320000
_EJ = 160000
_HALF = _E // 2
_NC = 2
_NS = 16
_KC = 80
_KJ = 40
_NCK = 25
_SUPC = _KC * _NCK
_SUPJ = _KJ * _NCK
_NSUPC = _HALF // (_NS * _SUPC)
_NSUPJ = _EJ // (_NC * _NS * _SUPJ)
_RSTEP = 624
_RCNT = 640


def _pipe_pass(nsup, sup, nck, k, base0, idx_hbm, dst_hbm, table, acc_sp,
               rowsbufs, istgbufs, dstgbufs, semsI, semsG,
               scale_fn=None, wb_hbm=None, wbbufs=None):
  def sup_body(b, carry):
    base = base0 + b * sup
    desc_i = [[None, None, None], [None, None, None]]
    desc_g = [None, None]

    def fire_idx(j):
      p = j % 2
      off = pl.multiple_of(base + j * k, 8)
      desc_i[p][0] = pltpu.async_copy(idx_hbm.at[pl.ds(off, k)],
                                      istgbufs[p], semsI[p])
      desc_i[p][1] = pltpu.async_copy(dst_hbm.at[pl.ds(off, k)],
                                      dstgbufs[p], semsI[p])
      if wb_hbm is not None:
        desc_i[p][2] = pltpu.async_copy(wb_hbm.at[pl.ds(off, k)],
                                        wbbufs[p], semsI[p])

    def wait_idx(j):
      p = j % 2
      desc_i[p][0].wait()
      desc_i[p][1].wait()
      if wb_hbm is not None:
        desc_i[p][2].wait()

    def fire_gather(j):
      p = j % 2
      desc_g[p] = pltpu.async_copy(table.at[istgbufs[p]],
                                   rowsbufs[p].at[pl.ds(0, k)], semsG[p])

    fire_idx(0)
    wait_idx(0)
    fire_gather(0)
    if nck > 1:
      fire_idx(1)
    for j in range(nck):
      p = j % 2
      if j + 1 < nck:
        wait_idx(j + 1)
        fire_gather(j + 1)
      desc_g[p].wait()
      if scale_fn is not None:
        scale_fn(p)
      pltpu.sync_copy(rowsbufs[p].at[pl.ds(0, k)], acc_sp.at[dstgbufs[p]],
                      add=True)
      if j + 2 < nck:
        fire_idx(j + 2)
    return carry

  lax.fori_loop(0, nsup, sup_body, None)


def _sc_body(emb, negrel, srcc, typc, dstc, srcj, dstj, wbj,
             acc_in, acc_out, deg_in, deg_out, accj0, accj1,
             acc_sp, rows0, rows1, istg0, istg1, dstg0, dstg1,
             istgj0, istgj1, dstgj0, dstgj1, wbstg0, wbstg1, ones_v,
             semI0, semI1, semG0, semG1):
  c = lax.axis_index("c")
  s = lax.axis_index("s")
  row0 = pl.multiple_of(s * _RSTEP, 8)
  rowsbufs = [rows0, rows1]
  semsI = [semI0, semI1]
  semsG = [semG0, semG1]

  zero16 = jnp.zeros((16,), jnp.float32)
  lane = lax.iota(jnp.int32, 16)
  one0 = jnp.where(lane == 0, 1.0, 0.0).astype(jnp.float32)

  def _init_row(r, carry):
    ones_v[r, pl.ds(0, 16)] = one0
    for j in range(1, _D // 16):
      ones_v[r, pl.ds(16 * j, 16)] = zero16
    for j in range(_D // 16):
      rows0[r, pl.ds(16 * j, 16)] = zero16
    return carry

  def _zero_rows0(r, carry):
    for j in range(_D // 16):
      rows0[r, pl.ds(16 * j, 16)] = zero16
    return carry

  def _zero_acc():
    for b in range(_RCNT // _KC):
      pltpu.sync_copy(rows0, acc_sp.at[pl.ds(row0 + b * _KC, _KC)])

  lax.fori_loop(0, _KC, _init_row, None)
  _zero_acc()
  plsc.subcore_barrier()

  cbase = c * _HALF + s * (_NSUPC * _SUPC)
  _pipe_pass(_NSUPC, _SUPC, _NCK, _KC, cbase, srcc, dstc, emb, acc_sp,
             rowsbufs, [istg0, istg1], [dstg0, dstg1], semsI, semsG)
  _pipe_pass(_NSUPC, _SUPC, _NCK, _KC, cbase, typc, dstc, negrel, acc_sp,
             rowsbufs, [istg0, istg1], [dstg0, dstg1], semsI, semsG)
  plsc.subcore_barrier()

  @pl.when(c == 0)
  def _dump_in():
    pltpu.sync_copy(acc_sp.at[pl.ds(row0, _RCNT)],
                    acc_in.at[pl.ds(row0, _RCNT)])

  @pl.when(c == 1)
  def _dump_out():
    pltpu.sync_copy(acc_sp.at[pl.ds(row0, _RCNT)],
                    acc_out.at[pl.ds(row0, _RCNT)])

  plsc.subcore_barrier()

  lax.fori_loop(0, _KC, _zero_rows0, None)
  _zero_acc()
  plsc.subcore_barrier()

  def deg_sup(b, carry):
    base = cbase + b * _SUPC
    desc_d = [None, None]

    def fire_didx(j):
      p = j % 2
      off = pl.multiple_of(base + j * _KC, 8)
      desc_d[p] = pltpu.async_copy(dstc.at[pl.ds(off, _KC)],
                                   [dstg0, dstg1][p], semsI[p])

    fire_didx(0)
    fire_didx(1)
    for j in range(_NCK):
      p = j % 2
      desc_d[p].wait()
      pltpu.sync_copy(ones_v, acc_sp.at[[dstg0, dstg1][p]], add=True)
      if j + 2 < _NCK:
        fire_didx(j + 2)
    return carry

  lax.fori_loop(0, _NSUPC, deg_sup, None)
  plsc.subcore_barrier()

  @pl.when(c == 0)
  def _dump_deg_in():
    pltpu.sync_copy(acc_sp.at[pl.ds(row0, _RCNT)],
                    deg_in.at[pl.ds(row0, _RCNT)])

  @pl.when(c == 1)
  def _dump_deg_out():
    pltpu.sync_copy(acc_sp.at[pl.ds(row0, _RCNT)],
                    deg_out.at[pl.ds(row0, _RCNT)])

  plsc.subcore_barrier()

  lax.fori_loop(0, _KC, _zero_rows0, None)
  _zero_acc()
  plsc.subcore_barrier()

  jbase = (c * _NS + s) * (_NSUPJ * _SUPJ)
  wbbufs = [wbstg0, wbstg1]

  def jump_scale(p):
    def srow(r, carry2):
      wvec = wbbufs[p][r, :]
      for m in range(_D // 16):
        rowsbufs[p][r, pl.ds(16 * m, 16)] = (
            rowsbufs[p][r, pl.ds(16 * m, 16)] * wvec)
      return carry2

    lax.fori_loop(0, _KJ, srow, None)

  _pipe_pass(_NSUPJ, _SUPJ, _NCK, _KJ, jbase, srcj, dstj, emb, acc_sp,
             rowsbufs, [istgj0, istgj1], [dstgj0, dstgj1], semsI, semsG,
             scale_fn=jump_scale, wb_hbm=wbj, wbbufs=wbbufs)
  plsc.subcore_barrier()

  @pl.when(c == 0)
  def _dump_j0():
    pltpu.sync_copy(acc_sp.at[pl.ds(row0, _RCNT)],
                    accj0.at[pl.ds(row0, _RCNT)])

  @pl.when(c == 1)
  def _dump_j1():
    pltpu.sync_copy(acc_sp.at[pl.ds(row0, _RCNT)],
                    accj1.at[pl.ds(row0, _RCNT)])


_sc_gather_scatter = functools.partial(
    pl.kernel,
    out_type=(
        jax.ShapeDtypeStruct((_N, _D), jnp.float32),
        jax.ShapeDtypeStruct((_N, _D), jnp.float32),
        jax.ShapeDtypeStruct((_N, _D), jnp.float32),
        jax.ShapeDtypeStruct((_N, _D), jnp.float32),
        jax.ShapeDtypeStruct((_N, _D), jnp.float32),
        jax.ShapeDtypeStruct((_N, _D), jnp.float32),
    ),
    mesh=plsc.VectorSubcoreMesh(core_axis_name="c", subcore_axis_name="s"),
    scratch_types=[
        pltpu.VMEM_SHARED((_N, _D), jnp.float32),
        pltpu.VMEM((_KC, _D), jnp.float32),
        pltpu.VMEM((_KC, _D), jnp.float32),
        pltpu.VMEM((_KC,), jnp.int32),
        pltpu.VMEM((_KC,), jnp.int32),
        pltpu.VMEM((_KC,), jnp.int32),
        pltpu.VMEM((_KC,), jnp.int32),
        pltpu.VMEM((_KJ,), jnp.int32),
        pltpu.VMEM((_KJ,), jnp.int32),
        pltpu.VMEM((_KJ,), jnp.int32),
        pltpu.VMEM((_KJ,), jnp.int32),
        pltpu.VMEM((_KJ, 16), jnp.float32),
        pltpu.VMEM((_KJ, 16), jnp.float32),
        pltpu.VMEM((_KC, _D), jnp.float32),
        pltpu.SemaphoreType.DMA,
        pltpu.SemaphoreType.DMA,
        pltpu.SemaphoreType.DMA,
        pltpu.SemaphoreType.DMA,
    ],
)(_sc_body)


_BLK = 2000


def _tc_body(acc_in, acc_out, deg_in, deg_out, aj0, aj1, emb, lrel,
             w_in, w_out, w_loop, w_jump, jw, out):
  ni = 1.0 / jnp.maximum(deg_in[:, 0:1], 1.0)
  no = 1.0 / jnp.maximum(deg_out[:, 0:1], 1.0)
  h = (jnp.dot(acc_in[...] * ni, w_in[...], preferred_element_type=jnp.float32)
       + jnp.dot(acc_out[...] * no, w_out[...],
                 preferred_element_type=jnp.float32)
       + jnp.dot(emb[...] - lrel[...], w_loop[...],
                 preferred_element_type=jnp.float32)) * (1.0 / 3.0)
  jr = jnp.dot(aj0[...] + aj1[...], w_jump[...],
               preferred_element_type=jnp.float32)
  out[...] = jnp.tanh(h) + jw[0, 0] * jr


def _tc_combine(acc_in, acc_out, deg_in, deg_out, aj0, aj1, emb, lrel,
                w_in, w_out, w_loop, w_jump, jw):
  row = pl.BlockSpec((_BLK, _D), lambda i: (i, 0))
  mat = pl.BlockSpec((_D, _D), lambda i: (0, 0))
  return pl.pallas_call(
      _tc_body,
      grid=(_N // _BLK,),
      in_specs=[row, row, row, row, row, row, row,
                pl.BlockSpec((1, _D), lambda i: (0, 0)),
                mat, mat, mat, mat,
                pl.BlockSpec((1, 1), lambda i: (0, 0))],
      out_specs=row,
      out_shape=jax.ShapeDtypeStruct((_N, _D), jnp.float32),
  )(acc_in, acc_out, deg_in, deg_out, aj0, aj1, emb, lrel,
    w_in, w_out, w_loop, w_jump, jw)


def kernel(t, emb, change, rel_emb, W_in, W_out, W_loop, loop_rel, W_jump,
           jump_weight, edge_w_jump, edge_index, edge_type, edge_id_jump):
  srcc = edge_index[0]
  dstc = edge_index[1]
  srcj = edge_id_jump[0]
  dstj = edge_id_jump[1]
  wbj = jnp.broadcast_to(edge_w_jump, (_EJ, 16))
  negrel = -rel_emb
  acc_in, acc_out, deg_in, deg_out, aj0, aj1 = _sc_gather_scatter(
      emb, negrel, srcc, edge_type, dstc, srcj, dstj, wbj)
  dchange = _tc_combine(acc_in, acc_out, deg_in, deg_out, aj0, aj1,
                        emb, loop_rel, W_in, W_out, W_loop, W_jump,
                        jump_weight.reshape(1, 1))
  return (change, dchange)


# speedup vs baseline: 4.3075x; 1.0309x over previous
import functools

import jax
import jax.numpy as jnp
from jax import lax
from jax.experimental import pallas as pl
from jax.experimental.pallas import tpu as pltpu
from jax.experimental.pallas import tpu_sc as plsc

_N = 10000
_D = 128
_E = 320000
_EJ = 160000
_HALF = _E // 2
_NC = 2
_NS = 16
_KC = 80
_KJ = 40
_NCK = 25
_SUPC = _KC * _NCK
_SUPJ = _KJ * _NCK
_NSUPC = _HALF // (_NS * _SUPC)
_NSUPJ = _EJ // (_NC * _NS * _SUPJ)
_RSTEP = 624
_RCNT = 640


def _pipe_pass(nsup, sup, nck, k, base0, idx_hbm, dst_hbm, table, acc_sp,
               rowsbufs, istgbufs, dstgbufs, semsI, semsG,
               scale_fn=None, wb_hbm=None, wbbufs=None):
  def sup_body(b, carry):
    base = base0 + b * sup
    desc_i = [[None, None, None], [None, None, None]]
    desc_g = [None, None]

    def fire_idx(j):
      p = j % 2
      off = pl.multiple_of(base + j * k, 8)
      desc_i[p][0] = pltpu.async_copy(idx_hbm.at[pl.ds(off, k)],
                                      istgbufs[p], semsI[p])
      desc_i[p][1] = pltpu.async_copy(dst_hbm.at[pl.ds(off, k)],
                                      dstgbufs[p], semsI[p])
      if wb_hbm is not None:
        desc_i[p][2] = pltpu.async_copy(wb_hbm.at[pl.ds(off, k)],
                                        wbbufs[p], semsI[p])

    def wait_idx(j):
      p = j % 2
      desc_i[p][0].wait()
      desc_i[p][1].wait()
      if wb_hbm is not None:
        desc_i[p][2].wait()

    def fire_gather(j):
      p = j % 2
      desc_g[p] = pltpu.async_copy(table.at[istgbufs[p]],
                                   rowsbufs[p].at[pl.ds(0, k)], semsG[p])

    fire_idx(0)
    wait_idx(0)
    fire_gather(0)
    if nck > 1:
      fire_idx(1)
    for j in range(nck):
      p = j % 2
      if j + 1 < nck:
        wait_idx(j + 1)
        fire_gather(j + 1)
      desc_g[p].wait()
      if scale_fn is not None:
        scale_fn(p)
      pltpu.sync_copy(rowsbufs[p].at[pl.ds(0, k)], acc_sp.at[dstgbufs[p]],
                      add=True)
      if j + 2 < nck:
        fire_idx(j + 2)
    return carry

  lax.fori_loop(0, nsup, sup_body, None)


def _sc_body(emb, negrel, srcc, typc, dstc, srcj, dstj, wbj,
             acc_in, acc_out, deg_in, deg_out, accj0, accj1,
             acc_sp, negrel_sp, rows0, rows1, istg0, istg1, dstg0, dstg1,
             istgj0, istgj1, dstgj0, dstgj1, wbstg0, wbstg1, ones_v,
             semI0, semI1, semG0, semG1):
  c = lax.axis_index("c")
  s = lax.axis_index("s")
  row0 = pl.multiple_of(s * _RSTEP, 8)
  rowsbufs = [rows0, rows1]
  semsI = [semI0, semI1]
  semsG = [semG0, semG1]

  zero16 = jnp.zeros((16,), jnp.float32)
  lane = lax.iota(jnp.int32, 16)
  one0 = jnp.where(lane == 0, 1.0, 0.0).astype(jnp.float32)

  def _init_row(r, carry):
    ones_v[r, pl.ds(0, 16)] = one0
    for j in range(1, _D // 16):
      ones_v[r, pl.ds(16 * j, 16)] = zero16
    for j in range(_D // 16):
      rows0[r, pl.ds(16 * j, 16)] = zero16
    return carry

  def _zero_rows0(r, carry):
    for j in range(_D // 16):
      rows0[r, pl.ds(16 * j, 16)] = zero16
    return carry

  def _zero_acc():
    for b in range(_RCNT // _KC):
      pltpu.sync_copy(rows0, acc_sp.at[pl.ds(row0 + b * _KC, _KC)])

  lax.fori_loop(0, _KC, _init_row, None)

  @pl.when(s == 0)
  def _stage_negrel():
    pltpu.sync_copy(negrel, negrel_sp)

  _zero_acc()
  plsc.subcore_barrier()

  cbase = c * _HALF + s * (_NSUPC * _SUPC)
  _pipe_pass(_NSUPC, _SUPC, _NCK, _KC, cbase, srcc, dstc, emb, acc_sp,
             rowsbufs, [istg0, istg1], [dstg0, dstg1], semsI, semsG)
  _pipe_pass(_NSUPC, _SUPC, _NCK, _KC, cbase, typc, dstc, negrel_sp, acc_sp,
             rowsbufs, [istg0, istg1], [dstg0, dstg1], semsI, semsG)
  plsc.subcore_barrier()

  @pl.when(c == 0)
  def _dump_in():
    pltpu.sync_copy(acc_sp.at[pl.ds(row0, _RCNT)],
                    acc_in.at[pl.ds(row0, _RCNT)])

  @pl.when(c == 1)
  def _dump_out():
    pltpu.sync_copy(acc_sp.at[pl.ds(row0, _RCNT)],
                    acc_out.at[pl.ds(row0, _RCNT)])

  plsc.subcore_barrier()

  lax.fori_loop(0, _KC, _zero_rows0, None)
  _zero_acc()
  plsc.subcore_barrier()

  def deg_sup(b, carry):
    base = cbase + b * _SUPC
    desc_d = [None, None]

    def fire_didx(j):
      p = j % 2
      off = pl.multiple_of(base + j * _KC, 8)
      desc_d[p] = pltpu.async_copy(dstc.at[pl.ds(off, _KC)],
                                   [dstg0, dstg1][p], semsI[p])

    fire_didx(0)
    fire_didx(1)
    for j in range(_NCK):
      p = j % 2
      desc_d[p].wait()
      pltpu.sync_copy(ones_v, acc_sp.at[[dstg0, dstg1][p]], add=True)
      if j + 2 < _NCK:
        fire_didx(j + 2)
    return carry

  lax.fori_loop(0, _NSUPC, deg_sup, None)
  plsc.subcore_barrier()

  @pl.when(c == 0)
  def _dump_deg_in():
    pltpu.sync_copy(acc_sp.at[pl.ds(row0, _RCNT)],
                    deg_in.at[pl.ds(row0, _RCNT)])

  @pl.when(c == 1)
  def _dump_deg_out():
    pltpu.sync_copy(acc_sp.at[pl.ds(row0, _RCNT)],
                    deg_out.at[pl.ds(row0, _RCNT)])

  plsc.subcore_barrier()

  lax.fori_loop(0, _KC, _zero_rows0, None)
  _zero_acc()
  plsc.subcore_barrier()

  jbase = (c * _NS + s) * (_NSUPJ * _SUPJ)
  wbbufs = [wbstg0, wbstg1]

  def jump_scale(p):
    def srow(r, carry2):
      wvec = wbbufs[p][r, :]
      for m in range(_D // 16):
        rowsbufs[p][r, pl.ds(16 * m, 16)] = (
            rowsbufs[p][r, pl.ds(16 * m, 16)] * wvec)
      return carry2

    lax.fori_loop(0, _KJ, srow, None)

  _pipe_pass(_NSUPJ, _SUPJ, _NCK, _KJ, jbase, srcj, dstj, emb, acc_sp,
             rowsbufs, [istgj0, istgj1], [dstgj0, dstgj1], semsI, semsG,
             scale_fn=jump_scale, wb_hbm=wbj, wbbufs=wbbufs)
  plsc.subcore_barrier()

  @pl.when(c == 0)
  def _dump_j0():
    pltpu.sync_copy(acc_sp.at[pl.ds(row0, _RCNT)],
                    accj0.at[pl.ds(row0, _RCNT)])

  @pl.when(c == 1)
  def _dump_j1():
    pltpu.sync_copy(acc_sp.at[pl.ds(row0, _RCNT)],
                    accj1.at[pl.ds(row0, _RCNT)])


_sc_gather_scatter = functools.partial(
    pl.kernel,
    out_type=(
        jax.ShapeDtypeStruct((_N, _D), jnp.float32),
        jax.ShapeDtypeStruct((_N, _D), jnp.float32),
        jax.ShapeDtypeStruct((_N, _D), jnp.float32),
        jax.ShapeDtypeStruct((_N, _D), jnp.float32),
        jax.ShapeDtypeStruct((_N, _D), jnp.float32),
        jax.ShapeDtypeStruct((_N, _D), jnp.float32),
    ),
    mesh=plsc.VectorSubcoreMesh(core_axis_name="c", subcore_axis_name="s"),
    scratch_types=[
        pltpu.VMEM_SHARED((_N, _D), jnp.float32),
        pltpu.VMEM_SHARED((200, _D), jnp.float32),
        pltpu.VMEM((_KC, _D), jnp.float32),
        pltpu.VMEM((_KC, _D), jnp.float32),
        pltpu.VMEM((_KC,), jnp.int32),
        pltpu.VMEM((_KC,), jnp.int32),
        pltpu.VMEM((_KC,), jnp.int32),
        pltpu.VMEM((_KC,), jnp.int32),
        pltpu.VMEM((_KJ,), jnp.int32),
        pltpu.VMEM((_KJ,), jnp.int32),
        pltpu.VMEM((_KJ,), jnp.int32),
        pltpu.VMEM((_KJ,), jnp.int32),
        pltpu.VMEM((_KJ, 16), jnp.float32),
        pltpu.VMEM((_KJ, 16), jnp.float32),
        pltpu.VMEM((_KC, _D), jnp.float32),
        pltpu.SemaphoreType.DMA,
        pltpu.SemaphoreType.DMA,
        pltpu.SemaphoreType.DMA,
        pltpu.SemaphoreType.DMA,
    ],
)(_sc_body)


_BLK = 2000


def _tc_body(acc_in, acc_out, deg_in, deg_out, aj0, aj1, emb, lrel,
             w_in, w_out, w_loop, w_jump, jw, out):
  ni = 1.0 / jnp.maximum(deg_in[:, 0:1], 1.0)
  no = 1.0 / jnp.maximum(deg_out[:, 0:1], 1.0)
  h = (jnp.dot(acc_in[...] * ni, w_in[...], preferred_element_type=jnp.float32)
       + jnp.dot(acc_out[...] * no, w_out[...],
                 preferred_element_type=jnp.float32)
       + jnp.dot(emb[...] - lrel[...], w_loop[...],
                 preferred_element_type=jnp.float32)) * (1.0 / 3.0)
  jr = jnp.dot(aj0[...] + aj1[...], w_jump[...],
               preferred_element_type=jnp.float32)
  out[...] = jnp.tanh(h) + jw[0, 0] * jr


def _tc_combine(acc_in, acc_out, deg_in, deg_out, aj0, aj1, emb, lrel,
                w_in, w_out, w_loop, w_jump, jw):
  row = pl.BlockSpec((_BLK, _D), lambda i: (i, 0))
  mat = pl.BlockSpec((_D, _D), lambda i: (0, 0))
  return pl.pallas_call(
      _tc_body,
      grid=(_N // _BLK,),
      in_specs=[row, row, row, row, row, row, row,
                pl.BlockSpec((1, _D), lambda i: (0, 0)),
                mat, mat, mat, mat,
                pl.BlockSpec((1, 1), lambda i: (0, 0))],
      out_specs=row,
      out_shape=jax.ShapeDtypeStruct((_N, _D), jnp.float32),
  )(acc_in, acc_out, deg_in, deg_out, aj0, aj1, emb, lrel,
    w_in, w_out, w_loop, w_jump, jw)


def kernel(t, emb, change, rel_emb, W_in, W_out, W_loop, loop_rel, W_jump,
           jump_weight, edge_w_jump, edge_index, edge_type, edge_id_jump):
  srcc = edge_index[0]
  dstc = edge_index[1]
  srcj = edge_id_jump[0]
  dstj = edge_id_jump[1]
  wbj = jnp.broadcast_to(edge_w_jump, (_EJ, 16))
  negrel = -rel_emb
  acc_in, acc_out, deg_in, deg_out, aj0, aj1 = _sc_gather_scatter(
      emb, negrel, srcc, edge_type, dstc, srcj, dstj, wbj)
  dchange = _tc_combine(acc_in, acc_out, deg_in, deg_out, aj0, aj1,
                        emb, loop_rel, W_in, W_out, W_loop, W_jump,
                        jump_weight.reshape(1, 1))
  return (change, dchange)
